# baseline (speedup 1.0000x reference)
import functools

import jax
import jax.numpy as jnp
from jax import lax
from jax.experimental import pallas as pl
from jax.experimental.pallas import tpu as pltpu
from jax.experimental.pallas import tpu_sc as plsc

N_TOKENS = 32768
EMB = 768
NE = 8
STEP = 4096
LANES = 16
NC = 2
NS = 16
NW = NC * NS
TOK_W = N_TOKENS // NW


def _dense_block(x_ref, w_ref, logitsT_ref, probsT_ref):
    x = x_ref[...]
    w = w_ref[...]
    lgT = lax.dot_general(w, x, (((0,), (1,)), ((), ())),
                          preferred_element_type=jnp.float32)
    logitsT_ref[...] = lgT
    mT = jnp.max(lgT, axis=0, keepdims=True)
    eT = jnp.exp(lgT - mT)
    probsT_ref[...] = eT / jnp.sum(eT, axis=0, keepdims=True)


def _dense_stage(noise_clock_emb, route_weight):
    grid = N_TOKENS // STEP
    return pl.pallas_call(
        _dense_block,
        grid=(grid,),
        in_specs=[
            pl.BlockSpec((STEP, EMB), lambda i: (i, 0)),
            pl.BlockSpec((EMB, NE), lambda i: (0, 0)),
        ],
        out_specs=[
            pl.BlockSpec((NE, STEP), lambda i: (0, i)),
            pl.BlockSpec((NE, STEP), lambda i: (0, i)),
        ],
        out_shape=(
            jax.ShapeDtypeStruct((NE, N_TOKENS), jnp.float32),
            jax.ShapeDtypeStruct((NE, N_TOKENS), jnp.float32),
        ),
    )(noise_clock_emb, route_weight)


def _route_body(probsT_hbm, idxT_hbm, wtsT_hbm,
                pbuf, i1buf, i2buf, w1buf, w2buf):
    wid = lax.axis_index("s") * NC + lax.axis_index("c")
    base = wid * TOK_W
    for e in range(NE):
        pltpu.sync_copy(probsT_hbm.at[e, pl.ds(base, TOK_W)],
                        pbuf.at[pl.ds(e * TOK_W, TOK_W)])

    def group(g, carry):
        off = g * LANES
        m1 = pbuf[pl.ds(off, LANES)]
        i1 = jnp.zeros((LANES,), jnp.int32)
        m2 = jnp.full((LANES,), -jnp.inf, jnp.float32)
        i2 = jnp.zeros((LANES,), jnp.int32)
        for e in range(1, NE):
            pe = pbuf[pl.ds(e * TOK_W + off, LANES)]
            ev = jnp.full((LANES,), e, jnp.int32)
            gt1 = pe > m1
            gt2 = pe > m2
            m2 = jnp.where(gt1, m1, jnp.where(gt2, pe, m2))
            i2 = jnp.where(gt1, i1, jnp.where(gt2, ev, i2))
            m1 = jnp.where(gt1, pe, m1)
            i1 = jnp.where(gt1, ev, i1)
        denom = jnp.maximum(m1 + m2, 1e-8)
        i1buf[pl.ds(off, LANES)] = i1
        i2buf[pl.ds(off, LANES)] = i2
        w1buf[pl.ds(off, LANES)] = m1 / denom
        w2buf[pl.ds(off, LANES)] = m2 / denom
        return carry

    lax.fori_loop(0, TOK_W // LANES, group, 0)

    pltpu.sync_copy(i1buf, idxT_hbm.at[0, pl.ds(base, TOK_W)])
    pltpu.sync_copy(i2buf, idxT_hbm.at[1, pl.ds(base, TOK_W)])
    pltpu.sync_copy(w1buf, wtsT_hbm.at[0, pl.ds(base, TOK_W)])
    pltpu.sync_copy(w2buf, wtsT_hbm.at[1, pl.ds(base, TOK_W)])


def _route_stage(probsT):
    mesh = plsc.VectorSubcoreMesh(core_axis_name="c", subcore_axis_name="s")
    k = pl.kernel(
        _route_body,
        mesh=mesh,
        out_type=[
            jax.ShapeDtypeStruct((2, N_TOKENS), jnp.int32),
            jax.ShapeDtypeStruct((2, N_TOKENS), jnp.float32),
        ],
        scratch_types=[
            pltpu.VMEM((NE * TOK_W,), jnp.float32),
            pltpu.VMEM((TOK_W,), jnp.int32),
            pltpu.VMEM((TOK_W,), jnp.int32),
            pltpu.VMEM((TOK_W,), jnp.float32),
            pltpu.VMEM((TOK_W,), jnp.float32),
        ],
    )
    return k(probsT)


@jax.jit
def kernel(noise_clock_emb, route_weight):
    logitsT, probsT = _dense_stage(noise_clock_emb, route_weight)
    idxT, wtsT = _route_stage(probsT)
    return (logitsT.T, probsT.T, idxT.T, wtsT.T)


# speedup vs baseline: 1.0443x; 1.0443x over previous
import functools

import jax
import jax.numpy as jnp
from jax import lax
from jax.experimental import pallas as pl
from jax.experimental.pallas import tpu as pltpu
from jax.experimental.pallas import tpu_sc as plsc

N_TOKENS = 32768
EMB = 768
NE = 8
STEP = 4096
LANES = 16
NC = 2
NS = 16
NW = NC * NS
TOK_W = N_TOKENS // NW


def _dense_block(x_ref, w_ref, logitsT_ref, probsT_ref):
    x = x_ref[...]
    w = w_ref[...]
    lgT = lax.dot_general(w, x, (((0,), (1,)), ((), ())),
                          preferred_element_type=jnp.float32)
    logitsT_ref[...] = lgT
    mT = jnp.max(lgT, axis=0, keepdims=True)
    eT = jnp.exp(lgT - mT)
    probsT_ref[...] = eT / jnp.sum(eT, axis=0, keepdims=True)


def _dense_stage(noise_clock_emb, route_weight):
    grid = N_TOKENS // STEP
    return pl.pallas_call(
        _dense_block,
        grid=(grid,),
        in_specs=[
            pl.BlockSpec((STEP, EMB), lambda i: (i, 0)),
            pl.BlockSpec((EMB, NE), lambda i: (0, 0)),
        ],
        out_specs=[
            pl.BlockSpec((NE, STEP), lambda i: (0, i)),
            pl.BlockSpec((NE, STEP), lambda i: (0, i)),
        ],
        out_shape=(
            jax.ShapeDtypeStruct((NE, N_TOKENS), jnp.float32),
            jax.ShapeDtypeStruct((NE, N_TOKENS), jnp.float32),
        ),
    )(noise_clock_emb, route_weight)


def _route_body(probsT_hbm, idxT_hbm, wtsT_hbm,
                pbuf, i1buf, i2buf, w1buf, w2buf):
    wid = lax.axis_index("s") * NC + lax.axis_index("c")
    base = wid * TOK_W
    pltpu.sync_copy(probsT_hbm.at[:, pl.ds(base, TOK_W)], pbuf)

    for g in range(TOK_W // LANES):
        off = g * LANES
        m1 = pbuf[0, pl.ds(off, LANES)]
        i1 = jnp.zeros((LANES,), jnp.int32)
        m2 = jnp.full((LANES,), -jnp.inf, jnp.float32)
        i2 = jnp.zeros((LANES,), jnp.int32)
        for e in range(1, NE):
            pe = pbuf[e, pl.ds(off, LANES)]
            ev = jnp.full((LANES,), e, jnp.int32)
            gt1 = pe > m1
            gt2 = pe > m2
            m2 = jnp.where(gt1, m1, jnp.where(gt2, pe, m2))
            i2 = jnp.where(gt1, i1, jnp.where(gt2, ev, i2))
            m1 = jnp.where(gt1, pe, m1)
            i1 = jnp.where(gt1, ev, i1)
        denom = jnp.maximum(m1 + m2, 1e-8)
        i1buf[pl.ds(off, LANES)] = i1
        i2buf[pl.ds(off, LANES)] = i2
        w1buf[pl.ds(off, LANES)] = m1 / denom
        w2buf[pl.ds(off, LANES)] = m2 / denom

    pltpu.sync_copy(i1buf, idxT_hbm.at[0, pl.ds(base, TOK_W)])
    pltpu.sync_copy(i2buf, idxT_hbm.at[1, pl.ds(base, TOK_W)])
    pltpu.sync_copy(w1buf, wtsT_hbm.at[0, pl.ds(base, TOK_W)])
    pltpu.sync_copy(w2buf, wtsT_hbm.at[1, pl.ds(base, TOK_W)])


def _route_stage(probsT):
    mesh = plsc.VectorSubcoreMesh(core_axis_name="c", subcore_axis_name="s")
    k = pl.kernel(
        _route_body,
        mesh=mesh,
        out_type=[
            jax.ShapeDtypeStruct((2, N_TOKENS), jnp.int32),
            jax.ShapeDtypeStruct((2, N_TOKENS), jnp.float32),
        ],
        scratch_types=[
            pltpu.VMEM((NE, TOK_W), jnp.float32),
            pltpu.VMEM((TOK_W,), jnp.int32),
            pltpu.VMEM((TOK_W,), jnp.int32),
            pltpu.VMEM((TOK_W,), jnp.float32),
            pltpu.VMEM((TOK_W,), jnp.float32),
        ],
    )
    return k(probsT)


@jax.jit
def kernel(noise_clock_emb, route_weight):
    logitsT, probsT = _dense_stage(noise_clock_emb, route_weight)
    idxT, wtsT = _route_stage(probsT)
    return (logitsT.T, probsT.T, idxT.T, wtsT.T)


# speedup vs baseline: 1.6360x; 1.5666x over previous
import functools

import jax
import jax.numpy as jnp
from jax import lax
from jax.experimental import pallas as pl

N_TOKENS = 32768
EMB = 768
NE = 8
NSPLIT = 1
Q = 4096
STEP = NSPLIT * Q


def _router_quarter(x, w, j, logitsT_ref, probsT_ref, idxT_ref, wtsT_ref):
    lgT = lax.dot_general(w, x, (((0,), (1,)), ((), ())),
                          preferred_element_type=jnp.float32)
    logitsT_ref[:, pl.ds(j * Q, Q)] = lgT
    mT = jnp.max(lgT, axis=0, keepdims=True)
    eT = jnp.exp(lgT - mT)
    pT = eT / jnp.sum(eT, axis=0, keepdims=True)
    probsT_ref[:, pl.ds(j * Q, Q)] = pT

    iota = lax.broadcasted_iota(jnp.int32, (NE, Q), 0)
    p1 = jnp.max(pT, axis=0, keepdims=True)
    i1 = jnp.min(jnp.where(pT == p1, iota, NE), axis=0, keepdims=True)
    masked = jnp.where(iota == i1, -jnp.inf, pT)
    p2 = jnp.max(masked, axis=0, keepdims=True)
    i2 = jnp.min(jnp.where(masked == p2, iota, NE), axis=0, keepdims=True)
    denom = jnp.maximum(p1 + p2, 1e-8)
    idxT_ref[:, pl.ds(j * Q, Q)] = jnp.concatenate([i1, i2], axis=0)
    wtsT_ref[:, pl.ds(j * Q, Q)] = jnp.concatenate(
        [p1 / denom, p2 / denom], axis=0)


def _router_block(*refs):
    x_refs = refs[:NSPLIT]
    w_ref = refs[NSPLIT]
    logitsT_ref, probsT_ref, idxT_ref, wtsT_ref = refs[NSPLIT + 1:]
    w = w_ref[...]
    for j in range(NSPLIT):
        _router_quarter(x_refs[j][...], w, j,
                        logitsT_ref, probsT_ref, idxT_ref, wtsT_ref)


@jax.jit
def kernel(noise_clock_emb, route_weight):
    grid = N_TOKENS // STEP

    def band(j):
        return pl.BlockSpec((Q, EMB), lambda i, j=j: (NSPLIT * i + j, 0))

    in_specs = [band(j) for j in range(NSPLIT)]
    in_specs.append(pl.BlockSpec((EMB, NE), lambda i: (0, 0)))
    out_specs = [
        pl.BlockSpec((NE, STEP), lambda i: (0, i)),
        pl.BlockSpec((NE, STEP), lambda i: (0, i)),
        pl.BlockSpec((2, STEP), lambda i: (0, i)),
        pl.BlockSpec((2, STEP), lambda i: (0, i)),
    ]
    out_shape = (
        jax.ShapeDtypeStruct((NE, N_TOKENS), jnp.float32),
        jax.ShapeDtypeStruct((NE, N_TOKENS), jnp.float32),
        jax.ShapeDtypeStruct((2, N_TOKENS), jnp.int32),
        jax.ShapeDtypeStruct((2, N_TOKENS), jnp.float32),
    )
    logitsT, probsT, idxT, wtsT = pl.pallas_call(
        _router_block,
        grid=(grid,),
        in_specs=in_specs,
        out_specs=out_specs,
        out_shape=out_shape,
    )(*([noise_clock_emb] * NSPLIT), route_weight)
    return (logitsT.T, probsT.T, idxT.T, wtsT.T)


# speedup vs baseline: 1.6640x; 1.0171x over previous
import jax
import jax.numpy as jnp
from jax import lax
from jax.experimental import pallas as pl

N_TOKENS = 32768
EMB = 768
NE = 8
STEP = 4096


def _router_block(x_ref, w_ref, logitsT_ref, probsT_ref, idxT_ref, wtsT_ref):
    x = x_ref[...]
    w = w_ref[...]
    lgT = lax.dot_general(w, x, (((0,), (1,)), ((), ())),
                          preferred_element_type=jnp.float32)
    logitsT_ref[...] = lgT
    mT = jnp.max(lgT, axis=0, keepdims=True)
    eT = jnp.exp(lgT - mT)
    pT = eT / jnp.sum(eT, axis=0, keepdims=True)
    probsT_ref[...] = pT

    iota = lax.broadcasted_iota(jnp.int32, (NE, STEP), 0)
    p1 = jnp.max(pT, axis=0, keepdims=True)
    i1 = jnp.min(jnp.where(pT == p1, iota, NE), axis=0, keepdims=True)
    masked = jnp.where(iota == i1, -jnp.inf, pT)
    p2 = jnp.max(masked, axis=0, keepdims=True)
    i2 = jnp.min(jnp.where(masked == p2, iota, NE), axis=0, keepdims=True)
    denom = jnp.maximum(p1 + p2, 1e-8)
    idxT_ref[...] = jnp.concatenate([i1, i2], axis=0)
    wtsT_ref[...] = jnp.concatenate([p1 / denom, p2 / denom], axis=0)


@jax.jit
def kernel(noise_clock_emb, route_weight):
    grid = N_TOKENS // STEP
    logitsT, probsT, idxT, wtsT = pl.pallas_call(
        _router_block,
        grid=(grid,),
        in_specs=[
            pl.BlockSpec((STEP, EMB), lambda i: (i, 0)),
            pl.BlockSpec((EMB, NE), lambda i: (0, 0)),
        ],
        out_specs=[
            pl.BlockSpec((NE, STEP), lambda i: (0, i)),
            pl.BlockSpec((NE, STEP), lambda i: (0, i)),
            pl.BlockSpec((2, STEP), lambda i: (0, i)),
            pl.BlockSpec((2, STEP), lambda i: (0, i)),
        ],
        out_shape=(
            jax.ShapeDtypeStruct((NE, N_TOKENS), jnp.float32),
            jax.ShapeDtypeStruct((NE, N_TOKENS), jnp.float32),
            jax.ShapeDtypeStruct((2, N_TOKENS), jnp.int32),
            jax.ShapeDtypeStruct((2, N_TOKENS), jnp.float32),
        ),
    )(noise_clock_emb, route_weight)
    return (logitsT.T, probsT.T, idxT.T, wtsT.T)
